# Initial kernel scaffold; baseline (speedup 1.0000x reference)
import functools

import jax
import jax.numpy as jnp
from jax import lax
from jax.experimental import pallas as pl
from jax.experimental.pallas import tpu as pltpu
from jax.experimental.pallas import tpu_sc as plsc

N = 10000
E = 320000
D = 128
H = 64
NUM_REL = 2

NC = 2
NS = 16
EPT = E // NS
CH = 80
G = 10
NSUP = EPT // (CH * G)
ROWS_CH = E // CH
RPT = (2 * N) // NS

_HIGH = jax.lax.Precision.HIGHEST


def _lrelu(v):
    return jnp.where(v >= 0, v, 0.01 * v)


def _dot(a, b):
    return jnp.dot(a, b, preferred_element_type=jnp.float32, precision=_HIGH)


def _prestage_body(nump_ref, catp_ref, src_ref, dst_ref, typ_ref,
                   wn_ref, bn_ref, wc_ref, bc_ref, wi_ref, bi_ref,
                   wr0_ref, wr1_ref, wroot_ref, brgcn_ref,
                   tab_ref, root_ref, idxs_ref, sdx_ref):
    n = _lrelu(_dot(nump_ref[...], wn_ref[...]) + bn_ref[...])
    c = _lrelu(_dot(catp_ref[...], wc_ref[...]) + bc_ref[...])
    x = jnp.concatenate((n, c), axis=1)
    x = _lrelu(_dot(x, wi_ref[...]) + bi_ref[...])
    xr0 = _dot(x, wr0_ref[...])
    xr1 = _dot(x, wr1_ref[...])
    tab_ref[0:N, :] = xr0[:, 0:64]
    tab_ref[N:2 * N, :] = xr1[:, 0:64]
    tab_ref[2 * N:3 * N, :] = xr0[:, 64:128]
    tab_ref[3 * N:4 * N, :] = xr1[:, 64:128]
    root_ref[...] = _dot(x, wroot_ref[...]) + brgcn_ref[...]
    idx = src_ref[...] + typ_ref[...] * N
    idxs_ref[0] = idx
    idxs_ref[1] = idx + 2 * N
    sdx_ref[...] = dst_ref[...] + typ_ref[...] * N


_prestage = pl.pallas_call(
    _prestage_body,
    out_shape=[
        jax.ShapeDtypeStruct((4 * N, 64), jnp.float32),
        jax.ShapeDtypeStruct((N, D), jnp.float32),
        jax.ShapeDtypeStruct((2, ROWS_CH, CH), jnp.int32),
        jax.ShapeDtypeStruct((ROWS_CH, CH), jnp.int32),
    ],
)


def _combine(acc, cnt, root):
    agg0 = jnp.concatenate((acc[0, 0:N, :], acc[1, 0:N, :]), axis=1)
    agg1 = jnp.concatenate((acc[0, N:2 * N, :], acc[1, N:2 * N, :]), axis=1)
    inv0 = 1.0 / jnp.maximum(cnt[0:N, 0:1], 1.0)
    inv1 = 1.0 / jnp.maximum(cnt[N:2 * N, 0:1], 1.0)
    return root + agg0 * inv0 + agg1 * inv1


def _mid_body(acc_ref, cnt_ref, root_ref, wr0_ref, wr1_ref, wroot_ref,
              brgcn_ref, tab_ref, root2_ref):
    x1 = _combine(acc_ref[...], cnt_ref[...], root_ref[...])
    xr0 = _dot(x1, wr0_ref[...])
    xr1 = _dot(x1, wr1_ref[...])
    tab_ref[0:N, :] = xr0[:, 0:64]
    tab_ref[N:2 * N, :] = xr1[:, 0:64]
    tab_ref[2 * N:3 * N, :] = xr0[:, 64:128]
    tab_ref[3 * N:4 * N, :] = xr1[:, 64:128]
    root2_ref[...] = _dot(x1, wroot_ref[...]) + brgcn_ref[...]


_mid = pl.pallas_call(
    _mid_body,
    out_shape=[
        jax.ShapeDtypeStruct((4 * N, 64), jnp.float32),
        jax.ShapeDtypeStruct((N, D), jnp.float32),
    ],
)


def _head_body(acc_ref, cnt_ref, root_ref, wo1_ref, bo1_ref, wo2_ref,
               bo2_ref, out_ref):
    x2 = _combine(acc_ref[...], cnt_ref[...], root_ref[...])
    h = _lrelu(_dot(x2, wo1_ref[...]) + bo1_ref[...])
    out_ref[...] = _dot(h, wo2_ref[...]) + bo2_ref[...]


_head = pl.pallas_call(
    _head_body,
    out_shape=[jax.ShapeDtypeStruct((N, D), jnp.float32)],
)


def _make_sc_layer(with_counts: bool):
    mesh = plsc.VectorSubcoreMesh(core_axis_name="c", subcore_axis_name="s")
    out_type = [jax.ShapeDtypeStruct((NC, 2 * N, 64), jnp.float32)]
    scratch = [
        pltpu.VMEM((G, CH), jnp.int32),
        pltpu.VMEM((G, CH), jnp.int32),
        pltpu.VMEM((CH, 64), jnp.float32),
        pltpu.VMEM((CH, 64), jnp.float32),
        pltpu.VMEM_SHARED((2 * N, 64), jnp.float32),
        pltpu.SemaphoreType.DMA,
        pltpu.SemaphoreType.DMA,
    ]
    if with_counts:
        out_type.append(jax.ShapeDtypeStruct((2 * N, 16), jnp.float32))
        scratch += [
            pltpu.VMEM((CH, 16), jnp.float32),
            pltpu.VMEM_SHARED((2 * N, 16), jnp.float32),
        ]

    def body(*refs):
        if with_counts:
            (idxs, sdxh, tab, z64, z16, onesh,
             acc_out, cnt_out,
             idx_v, sdx_v, buf0, buf1, acc_sh, sem0, sem1,
             ones_v, cnt_sh) = refs
        else:
            (idxs, sdxh, tab, z64,
             acc_out,
             idx_v, sdx_v, buf0, buf1, acc_sh, sem0, sem1) = refs

        c = lax.axis_index("c")
        s = lax.axis_index("s")
        r0 = s * RPT

        pltpu.sync_copy(z64, acc_sh.at[pl.ds(r0, RPT)])
        if with_counts:
            @pl.when(c == 0)
            def _zero_cnt():
                pltpu.sync_copy(z16, cnt_sh.at[pl.ds(r0, RPT)])
        plsc.subcore_barrier()

        if with_counts:
            @pl.when(c == 0)
            def _counts():
                pltpu.sync_copy(onesh, ones_v)

                def cnt_super(g, carry):
                    row = s * (EPT // CH) + g * G
                    pltpu.sync_copy(sdxh.at[pl.ds(row, G)], sdx_v)
                    for j in range(G):
                        pltpu.sync_copy(ones_v, cnt_sh.at[sdx_v.at[j]],
                                        add=True)
                    return carry

                lax.fori_loop(0, NSUP, cnt_super, 0)

        bufs = (buf0, buf1)
        sems = (sem0, sem1)

        def edge_super(g, carry):
            row = s * (EPT // CH) + g * G
            pltpu.sync_copy(idxs.at[c, pl.ds(row, G)], idx_v)
            pltpu.sync_copy(sdxh.at[pl.ds(row, G)], sdx_v)
            cps = [None, None]
            cps[0] = pltpu.async_copy(tab.at[idx_v.at[0]], bufs[0], sems[0])
            for j in range(G):
                b = j % 2
                if j + 1 < G:
                    cps[1 - b] = pltpu.async_copy(tab.at[idx_v.at[j + 1]],
                                                  bufs[1 - b], sems[1 - b])
                cps[b].wait()
                pltpu.sync_copy(bufs[b], acc_sh.at[sdx_v.at[j]], add=True)
            return carry

        lax.fori_loop(0, NSUP, edge_super, 0)

        plsc.subcore_barrier()
        pltpu.sync_copy(acc_sh.at[pl.ds(r0, RPT)],
                        acc_out.at[c, pl.ds(r0, RPT)])
        if with_counts:
            @pl.when(c == 0)
            def _cnt_out():
                pltpu.sync_copy(cnt_sh.at[pl.ds(r0, RPT)],
                                cnt_out.at[pl.ds(r0, RPT)])

    return pl.kernel(body, out_type=out_type, mesh=mesh,
                     scratch_types=scratch)


_sc_layer1 = _make_sc_layer(with_counts=True)
_sc_layer2 = _make_sc_layer(with_counts=False)


def kernel(des, tweet, num_prop, cat_prop, edge_index, edge_type,
           Wn, bn, Wc, bc, Wi, bi, Wrel, Wroot, brgcn, Wo1, bo1, Wo2, bo2):
    del des, tweet

    nump = jnp.pad(num_prop, ((0, 0), (0, 2)))
    catp = jnp.pad(cat_prop, ((0, 0), (0, 5)))
    wn = jnp.pad(Wn, ((0, 2), (0, 0)))
    wc = jnp.pad(Wc, ((0, 5), (0, 0)))
    wo2 = jnp.pad(Wo2, ((0, 0), (0, D - 2)))
    bo2p = jnp.pad(bo2, (0, D - 2)).reshape(1, D)
    src = edge_index[0].reshape(ROWS_CH, CH)
    dst = edge_index[1].reshape(ROWS_CH, CH)
    typ = edge_type.reshape(ROWS_CH, CH)
    z64 = jnp.zeros((RPT, 64), jnp.float32)
    z16 = jnp.zeros((RPT, 16), jnp.float32)
    ones = jnp.ones((CH, 16), jnp.float32)

    tab1, root1, idxs, sdx = _prestage(
        nump, catp, src, dst, typ,
        wn, bn.reshape(1, H), wc, bc.reshape(1, H), Wi, bi.reshape(1, D),
        Wrel[0], Wrel[1], Wroot, brgcn.reshape(1, D))

    acc1, cnt = _sc_layer1(idxs, sdx, tab1, z64, z16, ones)

    tab2, root2 = _mid(acc1, cnt, root1, Wrel[0], Wrel[1], Wroot,
                       brgcn.reshape(1, D))

    acc2 = _sc_layer2(idxs, sdx, tab2, z64)

    outp = _head(acc2, cnt, root2, Wo1, bo1.reshape(1, D), wo2, bo2p)
    return outp[:, 0:2]


# speedup vs baseline: 10.0828x; 10.0828x over previous
import functools

import jax
import jax.numpy as jnp
from jax import lax
from jax.experimental import pallas as pl
from jax.experimental.pallas import tpu as pltpu
from jax.experimental.pallas import tpu_sc as plsc

N = 10000
E = 320000
D = 128
H = 64
NUM_REL = 2

NC = 2
NS = 16
EPT = E // NS
CH = 80
NCHK = EPT // CH
G = 10
NSUP = NCHK // G
RPT = (2 * N) // NS

_HIGH = jax.lax.Precision.HIGHEST


def _lrelu(v):
    return jnp.where(v >= 0, v, 0.01 * v)


def _dot(a, b):
    return jnp.dot(a, b, preferred_element_type=jnp.float32)


BLK = 2000
GRID = N // BLK

_row = lambda i: (i, 0)
_fix = lambda i: (0, 0)


def _edges_body(src_ref, dst_ref, typ_ref, idxs_ref, sdx_ref):
    idx = src_ref[...] + typ_ref[...] * N
    idxs_ref[0] = idx
    idxs_ref[1] = idx + 2 * N
    sdx_ref[...] = dst_ref[...] + typ_ref[...] * N


_edges = pl.pallas_call(
    _edges_body,
    out_shape=[
        jax.ShapeDtypeStruct((2, E // D, D), jnp.int32),
        jax.ShapeDtypeStruct((E // D, D), jnp.int32),
    ],
)


def _write_tab(xr0, xr1, ta0_ref, ta1_ref, tb0_ref, tb1_ref):
    ta0_ref[...] = xr0[:, 0:64]
    ta1_ref[...] = xr1[:, 0:64]
    tb0_ref[...] = xr0[:, 64:128]
    tb1_ref[...] = xr1[:, 64:128]


def _prestage_body(nump_ref, catp_ref, wn_ref, bn_ref, wc_ref, bc_ref,
                   wi_ref, bi_ref, wr0_ref, wr1_ref, wroot_ref, brgcn_ref,
                   ta0_ref, ta1_ref, tb0_ref, tb1_ref, root_ref):
    n = _lrelu(_dot(nump_ref[...], wn_ref[...]) + bn_ref[...])
    c = _lrelu(_dot(catp_ref[...], wc_ref[...]) + bc_ref[...])
    x = jnp.concatenate((n, c), axis=1)
    x = _lrelu(_dot(x, wi_ref[...]) + bi_ref[...])
    _write_tab(_dot(x, wr0_ref[...]), _dot(x, wr1_ref[...]),
               ta0_ref, ta1_ref, tb0_ref, tb1_ref)
    root_ref[...] = _dot(x, wroot_ref[...]) + brgcn_ref[...]


_TAB_OUT = [jax.ShapeDtypeStruct((N, 64), jnp.float32)] * 4
_TAB_SPECS = [pl.BlockSpec((BLK, 64), _row)] * 4
_W_SPECS = [
    pl.BlockSpec((D, D), _fix),
    pl.BlockSpec((D, D), _fix),
    pl.BlockSpec((D, D), _fix),
    pl.BlockSpec((1, D), _fix),
]

_prestage = pl.pallas_call(
    _prestage_body,
    grid=(GRID,),
    in_specs=[
        pl.BlockSpec((BLK, 8), _row),
        pl.BlockSpec((BLK, 16), _row),
        pl.BlockSpec((8, H), _fix),
        pl.BlockSpec((1, H), _fix),
        pl.BlockSpec((16, H), _fix),
        pl.BlockSpec((1, H), _fix),
        pl.BlockSpec((D, D), _fix),
        pl.BlockSpec((1, D), _fix),
    ] + _W_SPECS,
    out_specs=_TAB_SPECS + [pl.BlockSpec((BLK, D), _row)],
    out_shape=_TAB_OUT + [jax.ShapeDtypeStruct((N, D), jnp.float32)],
)


def _combine(a00, a10, a01, a11, cnt0, cnt1, root):
    agg0 = jnp.concatenate((a00, a10), axis=1)
    agg1 = jnp.concatenate((a01, a11), axis=1)
    inv0 = 1.0 / jnp.maximum(cnt0[:, 0:1], 1.0)
    inv1 = 1.0 / jnp.maximum(cnt1[:, 0:1], 1.0)
    return root + agg0 * inv0 + agg1 * inv1


_ACC_SPECS = [
    pl.BlockSpec((1, BLK, 64), lambda i: (0, i, 0)),
    pl.BlockSpec((1, BLK, 64), lambda i: (1, i, 0)),
    pl.BlockSpec((1, BLK, 64), lambda i: (0, GRID + i, 0)),
    pl.BlockSpec((1, BLK, 64), lambda i: (1, GRID + i, 0)),
    pl.BlockSpec((BLK, 16), _row),
    pl.BlockSpec((BLK, 16), lambda i: (GRID + i, 0)),
    pl.BlockSpec((BLK, D), _row),
]


def _mid_body(a00_ref, a10_ref, a01_ref, a11_ref, cnt0_ref, cnt1_ref,
              root_ref, wr0_ref, wr1_ref, wroot_ref, brgcn_ref,
              ta0_ref, ta1_ref, tb0_ref, tb1_ref, root2_ref):
    x1 = _combine(a00_ref[0], a10_ref[0], a01_ref[0], a11_ref[0],
                  cnt0_ref[...], cnt1_ref[...], root_ref[...])
    _write_tab(_dot(x1, wr0_ref[...]), _dot(x1, wr1_ref[...]),
               ta0_ref, ta1_ref, tb0_ref, tb1_ref)
    root2_ref[...] = _dot(x1, wroot_ref[...]) + brgcn_ref[...]


_mid = pl.pallas_call(
    _mid_body,
    grid=(GRID,),
    in_specs=_ACC_SPECS + _W_SPECS,
    out_specs=_TAB_SPECS + [pl.BlockSpec((BLK, D), _row)],
    out_shape=_TAB_OUT + [jax.ShapeDtypeStruct((N, D), jnp.float32)],
)


def _head_body(a00_ref, a10_ref, a01_ref, a11_ref, cnt0_ref, cnt1_ref,
               root_ref, wo1_ref, bo1_ref, wo2_ref, bo2_ref, out_ref):
    x2 = _combine(a00_ref[0], a10_ref[0], a01_ref[0], a11_ref[0],
                  cnt0_ref[...], cnt1_ref[...], root_ref[...])
    h = _lrelu(_dot(x2, wo1_ref[...]) + bo1_ref[...])
    out_ref[...] = _dot(h, wo2_ref[...]) + bo2_ref[...]


_head = pl.pallas_call(
    _head_body,
    grid=(GRID,),
    in_specs=_ACC_SPECS + [
        pl.BlockSpec((D, D), _fix),
        pl.BlockSpec((1, D), _fix),
        pl.BlockSpec((D, D), _fix),
        pl.BlockSpec((1, D), _fix),
    ],
    out_specs=[pl.BlockSpec((BLK, D), _row)],
    out_shape=[jax.ShapeDtypeStruct((N, D), jnp.float32)],
)


def _make_sc_layer(with_counts: bool):
    mesh = plsc.VectorSubcoreMesh(core_axis_name="c", subcore_axis_name="s",
                                  num_cores=NC, num_subcores=NS)
    out_type = [jax.ShapeDtypeStruct((NC, NS, RPT, 64), jnp.float32)]
    scratch = [
        pltpu.VMEM((G, CH), jnp.int32),
        pltpu.VMEM((G, CH), jnp.int32),
        pltpu.VMEM((CH, 64), jnp.float32),
        pltpu.VMEM((CH, 64), jnp.float32),
        pltpu.VMEM_SHARED((2 * N, 64), jnp.float32),
        pltpu.SemaphoreType.DMA,
        pltpu.SemaphoreType.DMA,
    ]
    if with_counts:
        out_type.append(jax.ShapeDtypeStruct((NS, RPT, 16), jnp.float32))
        scratch += [
            pltpu.VMEM((CH, 16), jnp.float32),
            pltpu.VMEM_SHARED((2 * N, 16), jnp.float32),
        ]

    def body(*refs):
        if with_counts:
            (idxs, sdxh, tab, z64, z16, onesh,
             acc_out, cnt_out,
             idx_v, sdx_v, buf0, buf1, acc_sh, sem0, sem1,
             ones_v, cnt_sh) = refs
        else:
            (idxs, sdxh, tab, z64,
             acc_out,
             idx_v, sdx_v, buf0, buf1, acc_sh, sem0, sem1) = refs

        c = lax.axis_index("c")
        s = lax.axis_index("s")
        r0 = s * RPT

        pltpu.sync_copy(z64, acc_sh.at[pl.ds(r0, RPT)])
        if with_counts:
            @pl.when(c == 0)
            def _zero_cnt():
                pltpu.sync_copy(z16, cnt_sh.at[pl.ds(r0, RPT)])
        plsc.subcore_barrier()

        if with_counts:
            @pl.when(c == 0)
            def _counts():
                pltpu.sync_copy(onesh, ones_v)

                def cnt_super(g, carry):
                    row = s * NCHK + g * G
                    pltpu.sync_copy(sdxh.at[pl.ds(row, G)], sdx_v)
                    for j in range(G):
                        pltpu.sync_copy(ones_v, cnt_sh.at[sdx_v.at[j]],
                                        add=True)
                    return carry

                lax.fori_loop(0, NSUP, cnt_super, 0)

        bufs = (buf0, buf1)
        sems = (sem0, sem1)

        def edge_super(g, carry):
            row = s * NCHK + g * G
            pltpu.sync_copy(idxs.at[c, pl.ds(row, G)], idx_v)
            pltpu.sync_copy(sdxh.at[pl.ds(row, G)], sdx_v)
            cps = [None, None]
            cps[0] = pltpu.async_copy(tab.at[idx_v.at[0]], bufs[0], sems[0])
            for j in range(G):
                b = j % 2
                if j + 1 < G:
                    cps[1 - b] = pltpu.async_copy(tab.at[idx_v.at[j + 1]],
                                                  bufs[1 - b], sems[1 - b])
                cps[b].wait()
                pltpu.sync_copy(bufs[b], acc_sh.at[sdx_v.at[j]], add=True)
            return carry

        lax.fori_loop(0, NSUP, edge_super, 0)

        plsc.subcore_barrier()
        pltpu.sync_copy(acc_sh.at[pl.ds(r0, RPT)], acc_out.at[c, s])
        if with_counts:
            @pl.when(c == 0)
            def _cnt_out():
                pltpu.sync_copy(cnt_sh.at[pl.ds(r0, RPT)], cnt_out.at[s])

    return pl.kernel(
        body, out_type=out_type, mesh=mesh, scratch_types=scratch,
        compiler_params=pltpu.CompilerParams(use_tc_tiling_on_sc=False))


@functools.lru_cache(maxsize=None)
def _sc_layers():
    return _make_sc_layer(with_counts=True), _make_sc_layer(with_counts=False)


def kernel(des, tweet, num_prop, cat_prop, edge_index, edge_type,
           Wn, bn, Wc, bc, Wi, bi, Wrel, Wroot, brgcn, Wo1, bo1, Wo2, bo2):
    del des, tweet

    nump = jnp.pad(num_prop, ((0, 0), (0, 2)))
    catp = jnp.pad(cat_prop, ((0, 0), (0, 5)))
    wn = jnp.pad(Wn, ((0, 2), (0, 0)))
    wc = jnp.pad(Wc, ((0, 5), (0, 0)))
    wo2 = jnp.pad(Wo2, ((0, 0), (0, D - 2)))
    bo2p = jnp.pad(bo2, (0, D - 2)).reshape(1, D)
    src = edge_index[0].reshape(E // D, D)
    dst = edge_index[1].reshape(E // D, D)
    typ = edge_type.reshape(E // D, D)
    z64 = jnp.zeros((RPT, 64), jnp.float32)
    z16 = jnp.zeros((RPT, 16), jnp.float32)
    ones = jnp.ones((CH, 16), jnp.float32)

    idxs, sdx = _edges(src, dst, typ)
    idxs4 = idxs.reshape(2, E // CH, CH)
    sdx3 = sdx.reshape(E // CH, CH)

    ta0, ta1, tb0, tb1, root1 = _prestage(
        nump, catp, wn, bn.reshape(1, H), wc, bc.reshape(1, H),
        Wi, bi.reshape(1, D), Wrel[0], Wrel[1], Wroot, brgcn.reshape(1, D))
    tab1 = jnp.concatenate((ta0, ta1, tb0, tb1), axis=0)

    sc_layer1, sc_layer2 = _sc_layers()
    acc1, cnt = sc_layer1(idxs4, sdx3, tab1, z64, z16, ones)
    acc1 = acc1.reshape(NC, 2 * N, 64)
    cnt = cnt.reshape(2 * N, 16)

    ta0, ta1, tb0, tb1, root2 = _mid(
        acc1, acc1, acc1, acc1, cnt, cnt, root1,
        Wrel[0], Wrel[1], Wroot, brgcn.reshape(1, D))
    tab2 = jnp.concatenate((ta0, ta1, tb0, tb1), axis=0)

    (acc2,) = sc_layer2(idxs4, sdx3, tab2, z64)
    acc2 = acc2.reshape(NC, 2 * N, 64)

    (outp,) = _head(acc2, acc2, acc2, acc2, cnt, cnt, root2,
                    Wo1, bo1.reshape(1, D), wo2, bo2p)
    return outp[:, 0:2]


# speedup vs baseline: 10.8598x; 1.0771x over previous
import functools

import jax
import jax.numpy as jnp
from jax import lax
from jax.experimental import pallas as pl
from jax.experimental.pallas import tpu as pltpu
from jax.experimental.pallas import tpu_sc as plsc

N = 10000
E = 320000
D = 128
H = 64
NUM_REL = 2

NC = 2
NS = 16
EPT = E // NS
CH = 80
NCHK = EPT // CH
G = 10
NSUP = NCHK // G
RPT = (2 * N) // NS

_HIGH = jax.lax.Precision.HIGHEST


def _lrelu(v):
    return jnp.where(v >= 0, v, 0.01 * v)


def _dot(a, b):
    return jnp.dot(a, b, preferred_element_type=jnp.float32)


BLK = 2000
GRID = N // BLK

_row = lambda i: (i, 0)
_fix = lambda i: (0, 0)


EROWS, ECOLS = 1000, 320
EB = EROWS // GRID


def _write_tab(xr0, xr1, tab_ref):
    tab_ref[0] = xr0[:, 0:64]
    tab_ref[1] = xr1[:, 0:64]
    tab_ref[2] = xr0[:, 64:128]
    tab_ref[3] = xr1[:, 64:128]


def _prestage_body(nump_ref, catp_ref, src_ref, dst_ref, typ_ref,
                   wn_ref, bn_ref, wc_ref, bc_ref,
                   wi_ref, bi_ref, wr0_ref, wr1_ref, wroot_ref, brgcn_ref,
                   tab_ref, root_ref, idxs_ref, sdx_ref):
    n = _lrelu(_dot(nump_ref[...], wn_ref[...]) + bn_ref[...])
    c = _lrelu(_dot(catp_ref[...], wc_ref[...]) + bc_ref[...])
    x = jnp.concatenate((n, c), axis=1)
    x = _lrelu(_dot(x, wi_ref[...]) + bi_ref[...])
    _write_tab(_dot(x, wr0_ref[...]), _dot(x, wr1_ref[...]), tab_ref)
    root_ref[...] = _dot(x, wroot_ref[...]) + brgcn_ref[...]
    idx = src_ref[...] + typ_ref[...] * N
    idxs_ref[0] = idx
    idxs_ref[1] = idx + 2 * N
    sdx_ref[...] = dst_ref[...] + typ_ref[...] * N


_TAB_SPEC = pl.BlockSpec((4, BLK, 64), lambda i: (0, i, 0))
_TAB_OUT = jax.ShapeDtypeStruct((4, N, 64), jnp.float32)
_W_SPECS = [
    pl.BlockSpec((D, D), _fix),
    pl.BlockSpec((D, D), _fix),
    pl.BlockSpec((D, D), _fix),
    pl.BlockSpec((1, D), _fix),
]

_prestage = pl.pallas_call(
    _prestage_body,
    grid=(GRID,),
    in_specs=[
        pl.BlockSpec((BLK, 8), _row),
        pl.BlockSpec((BLK, 16), _row),
        pl.BlockSpec((EB, ECOLS), _row),
        pl.BlockSpec((EB, ECOLS), _row),
        pl.BlockSpec((EB, ECOLS), _row),
        pl.BlockSpec((8, H), _fix),
        pl.BlockSpec((1, H), _fix),
        pl.BlockSpec((16, H), _fix),
        pl.BlockSpec((1, H), _fix),
        pl.BlockSpec((D, D), _fix),
        pl.BlockSpec((1, D), _fix),
    ] + _W_SPECS,
    out_specs=[_TAB_SPEC, pl.BlockSpec((BLK, D), _row),
               pl.BlockSpec((2, EB, ECOLS), lambda i: (0, i, 0)),
               pl.BlockSpec((EB, ECOLS), _row)],
    out_shape=[_TAB_OUT, jax.ShapeDtypeStruct((N, D), jnp.float32),
               jax.ShapeDtypeStruct((2, EROWS, ECOLS), jnp.int32),
               jax.ShapeDtypeStruct((EROWS, ECOLS), jnp.int32)],
)


def _combine(a00, a10, a01, a11, cnt0, cnt1, root):
    agg0 = jnp.concatenate((a00, a10), axis=1)
    agg1 = jnp.concatenate((a01, a11), axis=1)
    inv0 = 1.0 / jnp.maximum(cnt0[:, 0:1], 1.0)
    inv1 = 1.0 / jnp.maximum(cnt1[:, 0:1], 1.0)
    return root + agg0 * inv0 + agg1 * inv1


_ACC_SPECS = [
    pl.BlockSpec((1, BLK, 64), lambda i: (0, i, 0)),
    pl.BlockSpec((1, BLK, 64), lambda i: (1, i, 0)),
    pl.BlockSpec((1, BLK, 64), lambda i: (0, GRID + i, 0)),
    pl.BlockSpec((1, BLK, 64), lambda i: (1, GRID + i, 0)),
    pl.BlockSpec((1, BLK, 16), lambda i: (0, i, 0)),
    pl.BlockSpec((1, BLK, 16), lambda i: (1, i, 0)),
    pl.BlockSpec((1, BLK, 16), lambda i: (0, GRID + i, 0)),
    pl.BlockSpec((1, BLK, 16), lambda i: (1, GRID + i, 0)),
    pl.BlockSpec((BLK, D), _row),
]


def _mid_body(a00_ref, a10_ref, a01_ref, a11_ref, c00_ref, c10_ref,
              c01_ref, c11_ref, root_ref, wr0_ref, wr1_ref, wroot_ref,
              brgcn_ref, tab_ref, root2_ref):
    x1 = _combine(a00_ref[0], a10_ref[0], a01_ref[0], a11_ref[0],
                  c00_ref[0] + c10_ref[0], c01_ref[0] + c11_ref[0],
                  root_ref[...])
    _write_tab(_dot(x1, wr0_ref[...]), _dot(x1, wr1_ref[...]), tab_ref)
    root2_ref[...] = _dot(x1, wroot_ref[...]) + brgcn_ref[...]


_mid = pl.pallas_call(
    _mid_body,
    grid=(GRID,),
    in_specs=_ACC_SPECS + _W_SPECS,
    out_specs=[_TAB_SPEC, pl.BlockSpec((BLK, D), _row)],
    out_shape=[_TAB_OUT, jax.ShapeDtypeStruct((N, D), jnp.float32)],
)


def _head_body(a00_ref, a10_ref, a01_ref, a11_ref, c00_ref, c10_ref,
               c01_ref, c11_ref, root_ref, wo1_ref, bo1_ref, wo2_ref,
               bo2_ref, out_ref):
    x2 = _combine(a00_ref[0], a10_ref[0], a01_ref[0], a11_ref[0],
                  c00_ref[0] + c10_ref[0], c01_ref[0] + c11_ref[0],
                  root_ref[...])
    h = _lrelu(_dot(x2, wo1_ref[...]) + bo1_ref[...])
    out_ref[...] = _dot(h, wo2_ref[...]) + bo2_ref[...]


_head = pl.pallas_call(
    _head_body,
    grid=(GRID,),
    in_specs=_ACC_SPECS + [
        pl.BlockSpec((D, D), _fix),
        pl.BlockSpec((1, D), _fix),
        pl.BlockSpec((D, D), _fix),
        pl.BlockSpec((1, D), _fix),
    ],
    out_specs=[pl.BlockSpec((BLK, D), _row)],
    out_shape=[jax.ShapeDtypeStruct((N, D), jnp.float32)],
)


def _make_sc_layer(with_counts: bool):
    mesh = plsc.VectorSubcoreMesh(core_axis_name="c", subcore_axis_name="s",
                                  num_cores=NC, num_subcores=NS)
    out_type = [jax.ShapeDtypeStruct((NC, NS, RPT, 64), jnp.float32)]
    scratch = [
        pltpu.VMEM((G, CH), jnp.int32),
        pltpu.VMEM((G, CH), jnp.int32),
        pltpu.VMEM((CH, 64), jnp.float32),
        pltpu.VMEM((CH, 64), jnp.float32),
        pltpu.VMEM_SHARED((2 * N, 64), jnp.float32),
        pltpu.SemaphoreType.DMA,
        pltpu.SemaphoreType.DMA,
    ]
    if with_counts:
        out_type.append(jax.ShapeDtypeStruct((NC, NS, RPT, 16), jnp.float32))
        scratch += [
            pltpu.VMEM((CH, 16), jnp.float32),
            pltpu.VMEM_SHARED((2 * N, 16), jnp.float32),
        ]

    def body(*refs):
        if with_counts:
            (idxs, sdxh, tab, z64, z16, onesh,
             acc_out, cnt_out,
             idx_v, sdx_v, buf0, buf1, acc_sh, sem0, sem1,
             ones_v, cnt_sh) = refs
        else:
            (idxs, sdxh, tab, z64,
             acc_out,
             idx_v, sdx_v, buf0, buf1, acc_sh, sem0, sem1) = refs

        c = lax.axis_index("c")
        s = lax.axis_index("s")
        r0 = s * RPT

        pltpu.sync_copy(z64, acc_sh.at[pl.ds(r0, RPT)])
        if with_counts:
            pltpu.sync_copy(z16, cnt_sh.at[pl.ds(r0, RPT)])
            pltpu.sync_copy(onesh, ones_v)
        plsc.subcore_barrier()

        bufs = (buf0, buf1)
        sems = (sem0, sem1)

        def edge_super(g, carry):
            row = s * NCHK + g * G
            pltpu.sync_copy(idxs.at[c, pl.ds(row, G)], idx_v)
            pltpu.sync_copy(sdxh.at[pl.ds(row, G)], sdx_v)
            cps = [None, None]
            cps[0] = pltpu.async_copy(tab.at[idx_v.at[0]], bufs[0], sems[0])
            for j in range(G):
                b = j % 2
                if j + 1 < G:
                    cps[1 - b] = pltpu.async_copy(tab.at[idx_v.at[j + 1]],
                                                  bufs[1 - b], sems[1 - b])
                cps[b].wait()
                pltpu.sync_copy(bufs[b], acc_sh.at[sdx_v.at[j]], add=True)
            if with_counts:
                @pl.when((g % NC) == c)
                def _counts():
                    for j in range(G):
                        pltpu.sync_copy(ones_v, cnt_sh.at[sdx_v.at[j]],
                                        add=True)
            return carry

        lax.fori_loop(0, NSUP, edge_super, 0)

        plsc.subcore_barrier()
        pltpu.sync_copy(acc_sh.at[pl.ds(r0, RPT)], acc_out.at[c, s])
        if with_counts:
            pltpu.sync_copy(cnt_sh.at[pl.ds(r0, RPT)], cnt_out.at[c, s])

    return pl.kernel(
        body, out_type=out_type, mesh=mesh, scratch_types=scratch,
        compiler_params=pltpu.CompilerParams(use_tc_tiling_on_sc=False))


@functools.lru_cache(maxsize=None)
def _sc_layers():
    return _make_sc_layer(with_counts=True), _make_sc_layer(with_counts=False)


def kernel(des, tweet, num_prop, cat_prop, edge_index, edge_type,
           Wn, bn, Wc, bc, Wi, bi, Wrel, Wroot, brgcn, Wo1, bo1, Wo2, bo2):
    del des, tweet

    nump = jnp.pad(num_prop, ((0, 0), (0, 2)))
    catp = jnp.pad(cat_prop, ((0, 0), (0, 5)))
    wn = jnp.pad(Wn, ((0, 2), (0, 0)))
    wc = jnp.pad(Wc, ((0, 5), (0, 0)))
    wo2 = jnp.pad(Wo2, ((0, 0), (0, D - 2)))
    bo2p = jnp.pad(bo2, (0, D - 2)).reshape(1, D)
    src = edge_index[0].reshape(EROWS, ECOLS)
    dst = edge_index[1].reshape(EROWS, ECOLS)
    typ = edge_type.reshape(EROWS, ECOLS)
    z64 = jnp.zeros((RPT, 64), jnp.float32)
    z16 = jnp.zeros((RPT, 16), jnp.float32)
    ones = jnp.ones((CH, 16), jnp.float32)

    tab1, root1, idxs, sdx = _prestage(
        nump, catp, src, dst, typ,
        wn, bn.reshape(1, H), wc, bc.reshape(1, H),
        Wi, bi.reshape(1, D), Wrel[0], Wrel[1], Wroot, brgcn.reshape(1, D))
    idxs4 = idxs.reshape(2, E // CH, CH)
    sdx3 = sdx.reshape(E // CH, CH)

    sc_layer1, sc_layer2 = _sc_layers()
    acc1, cnt = sc_layer1(idxs4, sdx3, tab1.reshape(4 * N, 64), z64, z16,
                          ones)
    acc1 = acc1.reshape(NC, 2 * N, 64)
    cnt = cnt.reshape(NC, 2 * N, 16)

    tab2, root2 = _mid(
        acc1, acc1, acc1, acc1, cnt, cnt, cnt, cnt, root1,
        Wrel[0], Wrel[1], Wroot, brgcn.reshape(1, D))

    (acc2,) = sc_layer2(idxs4, sdx3, tab2.reshape(4 * N, 64), z64)
    acc2 = acc2.reshape(NC, 2 * N, 64)

    (outp,) = _head(acc2, acc2, acc2, acc2, cnt, cnt, cnt, cnt, root2,
                    Wo1, bo1.reshape(1, D), wo2, bo2p)
    return outp[:, 0:2]


# speedup vs baseline: 12.4451x; 1.1460x over previous
import functools

import jax
import jax.numpy as jnp
from jax import lax
from jax.experimental import pallas as pl
from jax.experimental.pallas import tpu as pltpu
from jax.experimental.pallas import tpu_sc as plsc

N = 10000
E = 320000
D = 128
H = 64

NC = 2
NS = 16
EPT = E // NS
CH = 80
NCHK = EPT // CH
G = 10
NSUP = NCHK // G
RPT = (2 * N) // NS


def _lrelu(v):
    return jnp.where(v >= 0, v, 0.01 * v)


def _dot(a, b):
    return jnp.dot(a, b, preferred_element_type=jnp.float32)


BLK = 2000
GRID = N // BLK

_row = lambda i: (i, 0)
_fix = lambda i: (0, 0)


def _edges_body(src_ref, dst_ref, typ_ref, idxs_ref, sdx_ref, sdxb_ref):
    base = 2 * (src_ref[...] + typ_ref[...] * N)
    idxs_ref[0] = base
    idxs_ref[1] = base + 1
    sdx_ref[...] = 2 * dst_ref[...] + typ_ref[...]
    sdxb_ref[...] = dst_ref[...] + typ_ref[...] * N


_edges = pl.pallas_call(
    _edges_body,
    out_shape=[
        jax.ShapeDtypeStruct((2, E // D, D), jnp.int32),
        jax.ShapeDtypeStruct((E // D, D), jnp.int32),
        jax.ShapeDtypeStruct((E // D, D), jnp.int32),
    ],
)


def _prestage_body(nump_ref, catp_ref, wn_ref, bn_ref, wc_ref, bc_ref,
                   wi_ref, bi_ref, wr0_ref, wr1_ref, wroot_ref, brgcn_ref,
                   tab_ref, root_ref):
    n = _lrelu(_dot(nump_ref[...], wn_ref[...]) + bn_ref[...])
    c = _lrelu(_dot(catp_ref[...], wc_ref[...]) + bc_ref[...])
    x = jnp.concatenate((n, c), axis=1)
    x = _lrelu(_dot(x, wi_ref[...]) + bi_ref[...])
    tab_ref[0] = _dot(x, wr0_ref[...])
    tab_ref[1] = _dot(x, wr1_ref[...])
    root_ref[...] = _dot(x, wroot_ref[...]) + brgcn_ref[...]


_TAB_SPEC = pl.BlockSpec((2, BLK, D), lambda i: (0, i, 0))
_TAB_OUT = jax.ShapeDtypeStruct((2, N, D), jnp.float32)
_W_SPECS = [
    pl.BlockSpec((D, D), _fix),
    pl.BlockSpec((D, D), _fix),
    pl.BlockSpec((D, D), _fix),
    pl.BlockSpec((1, D), _fix),
]

_prestage = pl.pallas_call(
    _prestage_body,
    grid=(GRID,),
    in_specs=[
        pl.BlockSpec((BLK, 8), _row),
        pl.BlockSpec((BLK, 16), _row),
        pl.BlockSpec((8, H), _fix),
        pl.BlockSpec((1, H), _fix),
        pl.BlockSpec((16, H), _fix),
        pl.BlockSpec((1, H), _fix),
        pl.BlockSpec((D, D), _fix),
        pl.BlockSpec((1, D), _fix),
    ] + _W_SPECS,
    out_specs=[_TAB_SPEC, pl.BlockSpec((BLK, D), _row)],
    out_shape=[_TAB_OUT, jax.ShapeDtypeStruct((N, D), jnp.float32)],
)


def _combine(a0, a1, c00, c10, c01, c11, root):
    agg0 = jnp.concatenate((a0[:, 0:64], a1[:, 0:64]), axis=1)
    agg1 = jnp.concatenate((a0[:, 64:128], a1[:, 64:128]), axis=1)
    inv0 = 1.0 / jnp.maximum(c00[:, 0:1] + c10[:, 0:1], 1.0)
    inv1 = 1.0 / jnp.maximum(c01[:, 0:1] + c11[:, 0:1], 1.0)
    return root + agg0 * inv0 + agg1 * inv1


_ACC_SPECS = [
    pl.BlockSpec((1, BLK, D), lambda i: (0, i, 0)),
    pl.BlockSpec((1, BLK, D), lambda i: (1, i, 0)),
    pl.BlockSpec((1, BLK, 16), lambda i: (0, i, 0)),
    pl.BlockSpec((1, BLK, 16), lambda i: (1, i, 0)),
    pl.BlockSpec((1, BLK, 16), lambda i: (0, GRID + i, 0)),
    pl.BlockSpec((1, BLK, 16), lambda i: (1, GRID + i, 0)),
    pl.BlockSpec((BLK, D), _row),
]


def _mid_body(a0_ref, a1_ref, c00_ref, c10_ref, c01_ref, c11_ref, root_ref,
              wr0_ref, wr1_ref, wroot_ref, brgcn_ref, tab_ref, root2_ref):
    x1 = _combine(a0_ref[0], a1_ref[0], c00_ref[0], c10_ref[0],
                  c01_ref[0], c11_ref[0], root_ref[...])
    tab_ref[0] = _dot(x1, wr0_ref[...])
    tab_ref[1] = _dot(x1, wr1_ref[...])
    root2_ref[...] = _dot(x1, wroot_ref[...]) + brgcn_ref[...]


_mid = pl.pallas_call(
    _mid_body,
    grid=(GRID,),
    in_specs=_ACC_SPECS + _W_SPECS,
    out_specs=[_TAB_SPEC, pl.BlockSpec((BLK, D), _row)],
    out_shape=[_TAB_OUT, jax.ShapeDtypeStruct((N, D), jnp.float32)],
)


def _head_body(a0_ref, a1_ref, c00_ref, c10_ref, c01_ref, c11_ref, root_ref,
               wo1_ref, bo1_ref, wo2_ref, bo2_ref, out_ref):
    x2 = _combine(a0_ref[0], a1_ref[0], c00_ref[0], c10_ref[0],
                  c01_ref[0], c11_ref[0], root_ref[...])
    h = _lrelu(_dot(x2, wo1_ref[...]) + bo1_ref[...])
    out_ref[...] = _dot(h, wo2_ref[...]) + bo2_ref[...]


_head = pl.pallas_call(
    _head_body,
    grid=(GRID,),
    in_specs=_ACC_SPECS + [
        pl.BlockSpec((D, D), _fix),
        pl.BlockSpec((1, D), _fix),
        pl.BlockSpec((D, D), _fix),
        pl.BlockSpec((1, D), _fix),
    ],
    out_specs=[pl.BlockSpec((BLK, D), _row)],
    out_shape=[jax.ShapeDtypeStruct((N, D), jnp.float32)],
)


def _make_sc_layer(with_counts: bool):
    mesh = plsc.VectorSubcoreMesh(core_axis_name="c", subcore_axis_name="s",
                                  num_cores=NC, num_subcores=NS)
    out_type = [jax.ShapeDtypeStruct((NC, NS, RPT, 64), jnp.float32)]
    scratch = [
        pltpu.VMEM((G, CH), jnp.int32),
        pltpu.VMEM((G, CH), jnp.int32),
        pltpu.VMEM((CH, 64), jnp.float32),
        pltpu.VMEM((CH, 64), jnp.float32),
        pltpu.VMEM_SHARED((2 * N, 64), jnp.float32),
        pltpu.SemaphoreType.DMA,
        pltpu.SemaphoreType.DMA,
    ]
    if with_counts:
        out_type.append(jax.ShapeDtypeStruct((NC, NS, RPT, 16), jnp.float32))
        scratch += [
            pltpu.VMEM((G, CH), jnp.int32),
            pltpu.VMEM((CH, 16), jnp.float32),
            pltpu.VMEM_SHARED((2 * N, 16), jnp.float32),
        ]

    def body(*refs):
        if with_counts:
            (idxs, sdxh, sdxb, tab, z64, z16, onesh,
             acc_out, cnt_out,
             idx_v, sdx_v, buf0, buf1, acc_sh, sem0, sem1,
             sdxb_v, ones_v, cnt_sh) = refs
        else:
            (idxs, sdxh, tab, z64,
             acc_out,
             idx_v, sdx_v, buf0, buf1, acc_sh, sem0, sem1) = refs

        c = lax.axis_index("c")
        s = lax.axis_index("s")
        r0 = s * RPT

        pltpu.sync_copy(z64, acc_sh.at[pl.ds(r0, RPT)])
        if with_counts:
            pltpu.sync_copy(z16, cnt_sh.at[pl.ds(r0, RPT)])
            pltpu.sync_copy(onesh, ones_v)
        plsc.subcore_barrier()

        bufs = (buf0, buf1)
        sems = (sem0, sem1)

        def edge_super(g, carry):
            row = s * NCHK + g * G
            pltpu.sync_copy(idxs.at[c, pl.ds(row, G)], idx_v)
            pltpu.sync_copy(sdxh.at[pl.ds(row, G)], sdx_v)
            cps = [None, None]
            cps[0] = pltpu.async_copy(tab.at[idx_v.at[0]], bufs[0], sems[0])
            for j in range(G):
                b = j % 2
                if j + 1 < G:
                    cps[1 - b] = pltpu.async_copy(tab.at[idx_v.at[j + 1]],
                                                  bufs[1 - b], sems[1 - b])
                cps[b].wait()
                pltpu.sync_copy(bufs[b], acc_sh.at[sdx_v.at[j]], add=True)
            if with_counts:
                @pl.when((g % NC) == c)
                def _counts():
                    pltpu.sync_copy(sdxb.at[pl.ds(row, G)], sdxb_v)
                    for j in range(G):
                        pltpu.sync_copy(ones_v, cnt_sh.at[sdxb_v.at[j]],
                                        add=True)
            return carry

        lax.fori_loop(0, NSUP, edge_super, 0)

        plsc.subcore_barrier()
        pltpu.sync_copy(acc_sh.at[pl.ds(r0, RPT)], acc_out.at[c, s])
        if with_counts:
            pltpu.sync_copy(cnt_sh.at[pl.ds(r0, RPT)], cnt_out.at[c, s])

    return pl.kernel(
        body, out_type=out_type, mesh=mesh, scratch_types=scratch,
        compiler_params=pltpu.CompilerParams(use_tc_tiling_on_sc=False))


@functools.lru_cache(maxsize=None)
def _sc_layers():
    return _make_sc_layer(with_counts=True), _make_sc_layer(with_counts=False)


def kernel(des, tweet, num_prop, cat_prop, edge_index, edge_type,
           Wn, bn, Wc, bc, Wi, bi, Wrel, Wroot, brgcn, Wo1, bo1, Wo2, bo2):
    del des, tweet

    nump = jnp.pad(num_prop, ((0, 0), (0, 2)))
    catp = jnp.pad(cat_prop, ((0, 0), (0, 5)))
    wn = jnp.pad(Wn, ((0, 2), (0, 0)))
    wc = jnp.pad(Wc, ((0, 5), (0, 0)))
    wo2 = jnp.pad(Wo2, ((0, 0), (0, D - 2)))
    bo2p = jnp.pad(bo2, (0, D - 2)).reshape(1, D)
    src = edge_index[0].reshape(E // D, D)
    dst = edge_index[1].reshape(E // D, D)
    typ = edge_type.reshape(E // D, D)
    z64 = jnp.zeros((RPT, 64), jnp.float32)
    z16 = jnp.zeros((RPT, 16), jnp.float32)
    ones = jnp.ones((CH, 16), jnp.float32)

    idxs, sdx, sdxb = _edges(src, dst, typ)
    idxs4 = idxs.reshape(2, E // CH, CH)
    sdx3 = sdx.reshape(E // CH, CH)
    sdxb3 = sdxb.reshape(E // CH, CH)

    tab1, root1 = _prestage(
        nump, catp, wn, bn.reshape(1, H), wc, bc.reshape(1, H),
        Wi, bi.reshape(1, D), Wrel[0], Wrel[1], Wroot, brgcn.reshape(1, D))

    sc_layer1, sc_layer2 = _sc_layers()
    acc1, cnt = sc_layer1(idxs4, sdx3, sdxb3, tab1.reshape(4 * N, 64),
                          z64, z16, ones)
    acc1 = acc1.reshape(NC, N, D)
    cnt = cnt.reshape(NC, 2 * N, 16)

    tab2, root2 = _mid(acc1, acc1, cnt, cnt, cnt, cnt, root1,
                       Wrel[0], Wrel[1], Wroot, brgcn.reshape(1, D))

    (acc2,) = sc_layer2(idxs4, sdx3, tab2.reshape(4 * N, 64), z64)
    acc2 = acc2.reshape(NC, N, D)

    (outp,) = _head(acc2, acc2, cnt, cnt, cnt, cnt, root2,
                    Wo1, bo1.reshape(1, D), wo2, bo2p)
    return outp[:, 0:2]
